# Initial kernel scaffold; baseline (speedup 1.0000x reference)
import jax
import jax.numpy as jnp
from jax.experimental import pallas as pl


def kernel(x, batch, last_ixs, W1, b1, W2, b2, qw, qb):
    raise NotImplementedError("write your pallas kernel here")


# speedup vs baseline: 6.5045x; 6.5045x over previous
import functools
import jax
import jax.numpy as jnp
from jax import lax
from jax.experimental import pallas as pl
from jax.experimental.pallas import tpu as pltpu

B = 16
N = 32768
H = 128
BLK = 1024
NBLK = N // BLK

_NEG = -1e30


def _tc_body(x_ref, seg_ref, vi_ref, W1_ref, b12_ref, W2_ref, qw_ref, qb_ref,
             out_ref, u_sc, m_sc, s_sc, acc_sc):
    i = pl.program_id(0)

    @pl.when(i == 0)
    def _init():
        u_sc[...] = jnp.dot(vi_ref[...], W1_ref[...],
                            preferred_element_type=jnp.float32) + b12_ref[...]
        m_sc[...] = jnp.full((1, B), _NEG, jnp.float32)
        s_sc[...] = jnp.zeros((1, B), jnp.float32)
        acc_sc[...] = jnp.zeros((B, H), jnp.float32)

    x = x_ref[...]
    seg = seg_ref[...]
    iota = lax.broadcasted_iota(jnp.int32, (BLK, B), 1)
    ohb = seg == iota
    oh = ohb.astype(jnp.float32)

    z = jnp.dot(x, W2_ref[...], preferred_element_type=jnp.float32)
    z = z + jnp.dot(oh, u_sc[...], preferred_element_type=jnp.float32)
    h = jax.nn.sigmoid(z)
    lg = jnp.dot(h, qw_ref[...], preferred_element_type=jnp.float32) + qb_ref[...]

    m_old = m_sc[...]
    mb = jnp.max(jnp.where(ohb, lg, _NEG), axis=0, keepdims=True)
    m_new = jnp.maximum(m_old, mb)
    m_sc[...] = m_new

    mtok = jnp.sum(oh * m_new, axis=1, keepdims=True)
    ex = jnp.exp(lg - mtok)
    sum_b = jnp.sum(oh * ex, axis=0, keepdims=True)
    scale = jnp.exp(m_old - m_new)
    s_sc[...] = s_sc[...] * scale + sum_b

    eye = jnp.eye(B, dtype=jnp.float32)
    diag_scale = eye * scale
    accb = lax.dot_general(oh, ex * x, (((0,), (0,)), ((), ())),
                           preferred_element_type=jnp.float32)
    acc_sc[...] = jnp.dot(diag_scale, acc_sc[...],
                          preferred_element_type=jnp.float32) + accb

    @pl.when(i == NBLK - 1)
    def _fin():
        inv = 1.0 / (s_sc[...] + 1e-16)
        diag_inv = jnp.eye(B, dtype=jnp.float32) * inv
        out_ref[...] = jnp.dot(diag_inv, acc_sc[...],
                               preferred_element_type=jnp.float32)


@jax.jit
def kernel(x, batch, last_ixs, W1, b1, W2, b2, qw, qb):
    seg2 = batch.astype(jnp.int32).reshape(N, 1)
    vi = jnp.take(x, last_ixs, axis=0)
    b12 = (b1 + b2).reshape(1, H)
    qb2 = qb.reshape(1, 1)

    out = pl.pallas_call(
        _tc_body,
        grid=(NBLK,),
        in_specs=[
            pl.BlockSpec((BLK, H), lambda i: (i, 0)),
            pl.BlockSpec((BLK, 1), lambda i: (i, 0)),
            pl.BlockSpec((B, H), lambda i: (0, 0)),
            pl.BlockSpec((H, H), lambda i: (0, 0)),
            pl.BlockSpec((1, H), lambda i: (0, 0)),
            pl.BlockSpec((H, H), lambda i: (0, 0)),
            pl.BlockSpec((H, 1), lambda i: (0, 0)),
            pl.BlockSpec((1, 1), lambda i: (0, 0)),
        ],
        out_specs=pl.BlockSpec((B, H), lambda i: (0, 0)),
        out_shape=jax.ShapeDtypeStruct((B, H), jnp.float32),
        scratch_shapes=[
            pltpu.VMEM((B, H), jnp.float32),
            pltpu.VMEM((1, B), jnp.float32),
            pltpu.VMEM((1, B), jnp.float32),
            pltpu.VMEM((B, H), jnp.float32),
        ],
        compiler_params=pltpu.CompilerParams(
            dimension_semantics=("arbitrary",),
        ),
    )(x, seg2, vi, W1, b12, W2, qw, qb2)
    return out
